# baseline (speedup 1.0000x reference)
import functools

import jax
import jax.numpy as jnp
from jax import lax
from jax.experimental import pallas as pl
from jax.experimental.pallas import tpu as pltpu
from jax.experimental.pallas import tpu_sc as plsc

N_NODES = 100000
DIM = 128
NUM_GRAPHS = 512
NUM_CORES = 2
NUM_SUBCORES = 16
NW = NUM_CORES * NUM_SUBCORES

SCAT = 128
BLK = 256
NB_FULL = N_NODES // BLK
MAX_BLKS_PER_W = (NB_FULL + NW - 1) // NW
TAIL0 = NB_FULL * BLK
TAIL1 = TAIL0 + SCAT
TAIL = N_NODES - TAIL1
ROWS_PER_TILE = NUM_GRAPHS // NUM_SUBCORES
HIST = NUM_GRAPHS * 16


def _sc_pool_body(x_hbm, ids_hbm, sums_out, cnt_out,
                  rows0, idx0, rows1, idx1, zrow_v, trows_v, tidx_v,
                  cnt2_v, tmp_v, ocnt_v, in0, in1, acc_sh, slot_sh):
    cid = lax.axis_index("c")
    sid = lax.axis_index("s")
    wid = sid * NUM_CORES + cid
    lanes = lax.iota(jnp.int32, 16)
    one16 = jnp.full((16,), 1.0, dtype=jnp.float32)
    zero16 = jnp.zeros((16,), dtype=jnp.float32)
    sync = pltpu.sync_copy

    def zc(r, carry):
        plsc.store_scatter(cnt2_v, [r * 16 + lanes], zero16)
        return carry
    lax.fori_loop(0, NUM_GRAPHS, zc, 0)
    for r in range(ROWS_PER_TILE):
        for j in range(DIM // 16):
            zrow_v[r, pl.ds(j * 16, 16)] = zero16
    sync(zrow_v, acc_sh.at[pl.ds(sid * ROWS_PER_TILE, ROWS_PER_TILE), :])
    plsc.subcore_barrier()

    def start_in(i, idxb, rowsb, sem):
        b = wid + i * NW

        @pl.when(b < NB_FULL)
        def _():
            r0 = b * BLK
            pltpu.async_copy(ids_hbm.at[pl.ds(r0, SCAT)], idxb.at[0], sem)
            pltpu.async_copy(ids_hbm.at[pl.ds(r0 + SCAT, SCAT)], idxb.at[1],
                             sem)
            pltpu.async_copy(x_hbm.at[pl.ds(r0, BLK), :], rowsb, sem)

    def finish(i, idxb, rowsb, sem):
        b = wid + i * NW

        @pl.when(b < NB_FULL)
        def _():
            pltpu.make_async_copy(ids_hbm.at[pl.ds(0, SCAT)], idxb.at[0],
                                  sem).wait()
            pltpu.make_async_copy(ids_hbm.at[pl.ds(0, SCAT)], idxb.at[1],
                                  sem).wait()
            pltpu.make_async_copy(x_hbm.at[pl.ds(0, BLK), :], rowsb,
                                  sem).wait()
            sync(rowsb.at[pl.ds(0, SCAT), :], acc_sh.at[idxb.at[0]], add=True)
            sync(rowsb.at[pl.ds(SCAT, SCAT), :], acc_sh.at[idxb.at[1]],
                 add=True)
            for h in range(2):
                for q in range(SCAT // 16):
                    seg = idxb[h, pl.ds(q * 16, 16)]
                    plsc.addupdate_scatter(cnt2_v, [seg * 16 + lanes], one16)

    start_in(0, idx0, rows0, in0)

    def pair(j, carry):
        i0 = 2 * j
        start_in(i0 + 1, idx1, rows1, in1)
        finish(i0, idx0, rows0, in0)
        start_in(i0 + 2, idx0, rows0, in0)
        finish(i0 + 1, idx1, rows1, in1)
        return carry

    lax.fori_loop(0, (MAX_BLKS_PER_W + 1) // 2, pair, 0)

    @pl.when(wid == 0)
    def _():
        sync(ids_hbm.at[pl.ds(TAIL0, SCAT)], idx0.at[0])
        sync(x_hbm.at[pl.ds(TAIL0, SCAT), :], rows0.at[pl.ds(0, SCAT), :])
        sync(rows0.at[pl.ds(0, SCAT), :], acc_sh.at[idx0.at[0]], add=True)
        for q in range(SCAT // 16):
            seg = idx0[0, pl.ds(q * 16, 16)]
            plsc.addupdate_scatter(cnt2_v, [seg * 16 + lanes], one16)

    @pl.when(wid == 1)
    def _():
        sync(ids_hbm.at[pl.ds(TAIL1, TAIL)], tidx_v)
        sync(x_hbm.at[pl.ds(TAIL1, TAIL), :], trows_v)
        sync(trows_v, acc_sh.at[tidx_v], add=True)
        for q in range(TAIL // 16):
            seg = tidx_v[pl.ds(q * 16, 16)]
            plsc.addupdate_scatter(cnt2_v, [seg * 16 + lanes], one16)

    sync(cnt2_v, slot_sh.at[sid])
    plsc.subcore_barrier()

    base = cid * NUM_GRAPHS + sid * ROWS_PER_TILE
    sync(acc_sh.at[pl.ds(sid * ROWS_PER_TILE, ROWS_PER_TILE), :],
         sums_out.at[pl.ds(base, ROWS_PER_TILE), :])

    sync(slot_sh.at[:, pl.ds(sid * ROWS_PER_TILE * 16, ROWS_PER_TILE * 16)],
         tmp_v)
    for g in range(ROWS_PER_TILE):
        tot = tmp_v[0, pl.ds(g * 16, 16)]
        for t in range(1, NUM_SUBCORES):
            tot = tot + tmp_v[t, pl.ds(g * 16, 16)]
        ocnt_v[pl.ds(g * 16, 16)] = tot
    sync(ocnt_v, cnt_out.at[pl.ds((cid * NUM_GRAPHS + sid * ROWS_PER_TILE) * 16,
                                  ROWS_PER_TILE * 16)])


_sc_pool = functools.partial(
    pl.kernel,
    out_type=[
        jax.ShapeDtypeStruct((NUM_CORES * NUM_GRAPHS, DIM), jnp.float32),
        jax.ShapeDtypeStruct((NUM_CORES * NUM_GRAPHS * 16,), jnp.float32),
    ],
    mesh=plsc.VectorSubcoreMesh(core_axis_name="c", subcore_axis_name="s"),
    compiler_params=pltpu.CompilerParams(needs_layout_passes=False),
    scratch_types=[
        pltpu.VMEM((BLK, DIM), jnp.float32),
        pltpu.VMEM((2, SCAT), jnp.int32),
        pltpu.VMEM((BLK, DIM), jnp.float32),
        pltpu.VMEM((2, SCAT), jnp.int32),
        pltpu.VMEM((ROWS_PER_TILE, DIM), jnp.float32),
        pltpu.VMEM((TAIL, DIM), jnp.float32),
        pltpu.VMEM((TAIL,), jnp.int32),
        pltpu.VMEM((HIST,), jnp.float32),
        pltpu.VMEM((NUM_SUBCORES, ROWS_PER_TILE * 16), jnp.float32),
        pltpu.VMEM((ROWS_PER_TILE * 16,), jnp.float32),
        pltpu.SemaphoreType.DMA,
        pltpu.SemaphoreType.DMA,
        pltpu.VMEM_SHARED((NUM_GRAPHS, DIM), jnp.float32),
        pltpu.VMEM_SHARED((NUM_SUBCORES, HIST), jnp.float32),
    ],
)(_sc_pool_body)


def _mlp_body(sums_ref, cnt_ref, u_ref, w1_ref, b1_ref, w2_ref, b2_ref, out_ref):
    s = sums_ref[0:NUM_GRAPHS, :] + sums_ref[NUM_GRAPHS:2 * NUM_GRAPHS, :]
    c16 = cnt_ref[0:NUM_GRAPHS, :] + cnt_ref[NUM_GRAPHS:2 * NUM_GRAPHS, :]
    c = jnp.sum(c16, axis=1, keepdims=True)
    mean = s / jnp.maximum(c, 1.0)
    hp = jnp.dot(mean, w1_ref[0:DIM, :], preferred_element_type=jnp.float32)
    h = jnp.maximum(hp + u_ref[...] * w1_ref[DIM:DIM + 1, :] + b1_ref[...], 0.0)
    out_ref[...] = jnp.dot(h, w2_ref[...],
                           preferred_element_type=jnp.float32) + b2_ref[...]


def kernel(x, segment_ids, u, y, W1, b1, W2, b2):
    ids32 = segment_ids.astype(jnp.int32)
    sums2, counts2 = _sc_pool(x, ids32)
    counts2 = counts2.reshape(NUM_CORES * NUM_GRAPHS, 16)
    pred = pl.pallas_call(
        _mlp_body,
        out_shape=jax.ShapeDtypeStruct((NUM_GRAPHS, 1), jnp.float32),
    )(sums2, counts2, u, W1, b1.reshape(1, -1), W2, b2.reshape(1, 1))
    return (pred, y)


# speedup vs baseline: 1.0255x; 1.0255x over previous
import functools

import jax
import jax.numpy as jnp
from jax import lax
from jax.experimental import pallas as pl
from jax.experimental.pallas import tpu as pltpu
from jax.experimental.pallas import tpu_sc as plsc

N_NODES = 100000
DIM = 128
NUM_GRAPHS = 512
NUM_CORES = 2
NUM_SUBCORES = 16
NW = NUM_CORES * NUM_SUBCORES

BLK = 128
NB_FULL = N_NODES // BLK
TAIL = N_NODES - NB_FULL * BLK
MAX_BLKS_PER_W = (NB_FULL + NW - 1) // NW
ROWS_PER_TILE = NUM_GRAPHS // NUM_SUBCORES
HIST = NUM_GRAPHS * 16


def _sc_pool_body(x_hbm, ids_hbm, sums_out, cnt_out,
                  rows0, idx0, rows1, idx1, zrow_v, trows_v, tidx_v,
                  cnt2_v, tmp_v, ocnt_v, in0, in1, acc_sh, slot_sh):
    cid = lax.axis_index("c")
    sid = lax.axis_index("s")
    wid = sid * NUM_CORES + cid
    lanes = lax.iota(jnp.int32, 16)
    one16 = jnp.full((16,), 1.0, dtype=jnp.float32)
    zero16 = jnp.zeros((16,), dtype=jnp.float32)
    sync = pltpu.sync_copy

    def zc(r, carry):
        plsc.store_scatter(cnt2_v, [r * 16 + lanes], zero16)
        return carry
    lax.fori_loop(0, NUM_GRAPHS, zc, 0)
    for r in range(ROWS_PER_TILE):
        for j in range(DIM // 16):
            zrow_v[r, pl.ds(j * 16, 16)] = zero16
    sync(zrow_v, acc_sh.at[pl.ds(sid * ROWS_PER_TILE, ROWS_PER_TILE), :])
    plsc.subcore_barrier()

    def start_in(i, idxb, rowsb, sem):
        b = wid + i * NW

        @pl.when(b < NB_FULL)
        def _():
            r0 = b * BLK
            pltpu.async_copy(ids_hbm.at[pl.ds(r0, BLK)], idxb, sem)
            pltpu.async_copy(x_hbm.at[pl.ds(r0, BLK), :], rowsb, sem)

    def finish(i, idxb, rowsb, sem):
        b = wid + i * NW

        @pl.when(b < NB_FULL)
        def _():
            pltpu.make_async_copy(ids_hbm.at[pl.ds(0, BLK)], idxb, sem).wait()
            pltpu.make_async_copy(x_hbm.at[pl.ds(0, BLK), :], rowsb,
                                  sem).wait()
            sync(rowsb, acc_sh.at[idxb], add=True)
            for k in range(BLK // 16):
                seg = idxb[pl.ds(k * 16, 16)]
                plsc.addupdate_scatter(cnt2_v, [seg * 16 + lanes], one16)

    start_in(0, idx0, rows0, in0)

    def pair(j, carry):
        i0 = 2 * j
        start_in(i0 + 1, idx1, rows1, in1)
        finish(i0, idx0, rows0, in0)
        start_in(i0 + 2, idx0, rows0, in0)
        finish(i0 + 1, idx1, rows1, in1)
        return carry

    lax.fori_loop(0, (MAX_BLKS_PER_W + 1) // 2, pair, 0)

    @pl.when(wid == 0)
    def _():
        r0 = NB_FULL * BLK
        sync(ids_hbm.at[pl.ds(r0, TAIL)], tidx_v)
        sync(x_hbm.at[pl.ds(r0, TAIL), :], trows_v)
        sync(trows_v, acc_sh.at[tidx_v], add=True)
        for k in range(TAIL // 16):
            seg = tidx_v[pl.ds(k * 16, 16)]
            plsc.addupdate_scatter(cnt2_v, [seg * 16 + lanes], one16)

    sync(cnt2_v, slot_sh.at[sid])
    plsc.subcore_barrier()

    base = cid * NUM_GRAPHS + sid * ROWS_PER_TILE
    sync(acc_sh.at[pl.ds(sid * ROWS_PER_TILE, ROWS_PER_TILE), :],
         sums_out.at[pl.ds(base, ROWS_PER_TILE), :])

    sync(slot_sh.at[:, pl.ds(sid * ROWS_PER_TILE * 16, ROWS_PER_TILE * 16)],
         tmp_v)
    for g in range(ROWS_PER_TILE):
        tot = tmp_v[0, pl.ds(g * 16, 16)]
        for t in range(1, NUM_SUBCORES):
            tot = tot + tmp_v[t, pl.ds(g * 16, 16)]
        ocnt_v[pl.ds(g * 16, 16)] = tot
    sync(ocnt_v, cnt_out.at[pl.ds((cid * NUM_GRAPHS + sid * ROWS_PER_TILE) * 16,
                                  ROWS_PER_TILE * 16)])


_sc_pool = functools.partial(
    pl.kernel,
    out_type=[
        jax.ShapeDtypeStruct((NUM_CORES * NUM_GRAPHS, DIM), jnp.float32),
        jax.ShapeDtypeStruct((NUM_CORES * NUM_GRAPHS * 16,), jnp.float32),
    ],
    mesh=plsc.VectorSubcoreMesh(core_axis_name="c", subcore_axis_name="s"),
    compiler_params=pltpu.CompilerParams(needs_layout_passes=False),
    scratch_types=[
        pltpu.VMEM((BLK, DIM), jnp.float32),
        pltpu.VMEM((BLK,), jnp.int32),
        pltpu.VMEM((BLK, DIM), jnp.float32),
        pltpu.VMEM((BLK,), jnp.int32),
        pltpu.VMEM((ROWS_PER_TILE, DIM), jnp.float32),
        pltpu.VMEM((TAIL, DIM), jnp.float32),
        pltpu.VMEM((TAIL,), jnp.int32),
        pltpu.VMEM((HIST,), jnp.float32),
        pltpu.VMEM((NUM_SUBCORES, ROWS_PER_TILE * 16), jnp.float32),
        pltpu.VMEM((ROWS_PER_TILE * 16,), jnp.float32),
        pltpu.SemaphoreType.DMA,
        pltpu.SemaphoreType.DMA,
        pltpu.VMEM_SHARED((NUM_GRAPHS, DIM), jnp.float32),
        pltpu.VMEM_SHARED((NUM_SUBCORES, HIST), jnp.float32),
    ],
)(_sc_pool_body)


def _mlp_body(sums_ref, cnt_ref, u_ref, w1_ref, b1_ref, w2_ref, b2_ref, out_ref):
    s = sums_ref[0:NUM_GRAPHS, :] + sums_ref[NUM_GRAPHS:2 * NUM_GRAPHS, :]
    c16 = cnt_ref[0:NUM_GRAPHS, :] + cnt_ref[NUM_GRAPHS:2 * NUM_GRAPHS, :]
    c = jnp.sum(c16, axis=1, keepdims=True)
    mean = s / jnp.maximum(c, 1.0)
    hp = jnp.dot(mean, w1_ref[0:DIM, :], preferred_element_type=jnp.float32)
    h = jnp.maximum(hp + u_ref[...] * w1_ref[DIM:DIM + 1, :] + b1_ref[...], 0.0)
    out_ref[...] = jnp.dot(h, w2_ref[...],
                           preferred_element_type=jnp.float32) + b2_ref[...]


def kernel(x, segment_ids, u, y, W1, b1, W2, b2):
    ids32 = segment_ids.astype(jnp.int32)
    sums2, counts2 = _sc_pool(x, ids32)
    counts2 = counts2.reshape(NUM_CORES * NUM_GRAPHS, 16)
    pred = pl.pallas_call(
        _mlp_body,
        out_shape=jax.ShapeDtypeStruct((NUM_GRAPHS, 1), jnp.float32),
    )(sums2, counts2, u, W1, b1.reshape(1, -1), W2, b2.reshape(1, 1))
    return (pred, y)
